# baseline (speedup 1.0000x reference)
import functools

import jax
import jax.numpy as jnp
from jax.experimental import pallas as pl
from jax.experimental.pallas import tpu as pltpu

_BM = 400


def _pad2(x, rows, cols):
    return jnp.pad(x, ((0, rows - x.shape[0]), (0, cols - x.shape[1])))


def _layer1_body(adj_ref, g_ref, w1_ref, out_ref):
    a = jnp.dot(adj_ref[...], g_ref[...], preferred_element_type=jnp.float32)
    out_ref[...] = jnp.maximum(
        jnp.dot(a, w1_ref[...], preferred_element_type=jnp.float32), 0.0)


def _layer2_body(adj_ref, g_ref, w2_ref, w3_ref, out_ref):
    a = jnp.dot(adj_ref[...], g_ref[...], preferred_element_type=jnp.float32)
    h = jnp.maximum(
        jnp.dot(a, w2_ref[...], preferred_element_type=jnp.float32), 0.0)
    out_ref[...] = jnp.dot(h, w3_ref[...], preferred_element_type=jnp.float32)


def _layer3_body(adj_ref, g_ref, wo_ref, b_ref, out_ref):
    h = jnp.maximum(
        jnp.dot(adj_ref[...], g_ref[...], preferred_element_type=jnp.float32),
        0.0)
    out_ref[...] = jnp.maximum(
        jnp.dot(h, wo_ref[...], preferred_element_type=jnp.float32)
        + b_ref[...], 0.0)


def _strip_call(body, adj, g, consts, out_w):
    n = adj.shape[0]
    k = g.shape[0]
    in_specs = [
        pl.BlockSpec((_BM, k), lambda i: (i, 0)),
        pl.BlockSpec((k, g.shape[1]), lambda i: (0, 0)),
    ] + [pl.BlockSpec(c.shape, lambda i: (0,) * c.ndim) for c in consts]
    return pl.pallas_call(
        body,
        grid=(n // _BM,),
        in_specs=in_specs,
        out_specs=pl.BlockSpec((_BM, out_w), lambda i: (i, 0)),
        out_shape=jax.ShapeDtypeStruct((n, out_w), jnp.float32),
        compiler_params=pltpu.CompilerParams(
            dimension_semantics=("parallel",)),
    )(adj, g, *consts)


@functools.partial(jax.jit, static_argnames=())
def kernel(data, adj_m, W1, W2, W3, W_out, b_out):
    w1 = _pad2(W1, 128, 256)
    w2 = _pad2(W2, 256, 384)
    w3 = _pad2(W3, 384, 256)
    wo = _pad2(W_out, 256, 128)
    b = jnp.pad(b_out, (0, 128 - b_out.shape[0])).reshape(1, 128)

    h1 = _strip_call(_layer1_body, adj_m, data, (w1,), 256)
    g2 = _strip_call(_layer2_body, adj_m, h1, (w2, w3), 256)
    z = _strip_call(_layer3_body, adj_m, g2, (wo, b), 128)
    return z[:, :100]


# speedup vs baseline: 1.0662x; 1.0662x over previous
import functools

import jax
import jax.numpy as jnp
from jax.experimental import pallas as pl
from jax.experimental.pallas import tpu as pltpu

_N = 10000
_BM1 = 200
_BM = 400


def _pad2(x, rows, cols):
    return jnp.pad(x, ((0, rows - x.shape[0]), (0, cols - x.shape[1])))


def _layer1_body(adj_ref, g_ref, w1_ref, h1_ref, adjb_ref):
    adj_b = adj_ref[...].astype(jnp.bfloat16)
    adjb_ref[...] = adj_b
    a = jnp.dot(adj_b, g_ref[...], preferred_element_type=jnp.float32)
    h1 = jnp.maximum(
        jnp.dot(a.astype(jnp.bfloat16), w1_ref[...],
                preferred_element_type=jnp.float32), 0.0)
    h1_ref[...] = h1.astype(jnp.bfloat16)


def _layer2_body(adj_ref, g_ref, w2_ref, w3_ref, out_ref):
    a = jnp.dot(adj_ref[...], g_ref[...], preferred_element_type=jnp.float32)
    h = jnp.maximum(
        jnp.dot(a.astype(jnp.bfloat16), w2_ref[...],
                preferred_element_type=jnp.float32), 0.0)
    out_ref[...] = jnp.dot(
        h.astype(jnp.bfloat16), w3_ref[...],
        preferred_element_type=jnp.float32).astype(jnp.bfloat16)


def _layer3_body(adj_ref, g_ref, wo_ref, b_ref, out_ref):
    h = jnp.maximum(
        jnp.dot(adj_ref[...], g_ref[...], preferred_element_type=jnp.float32),
        0.0)
    out_ref[...] = jnp.maximum(
        jnp.dot(h.astype(jnp.bfloat16), wo_ref[...],
                preferred_element_type=jnp.float32) + b_ref[...], 0.0)


def _strip_call(body, bm, adj, g, consts, out_w, out_dtype):
    in_specs = [
        pl.BlockSpec((bm, _N), lambda i: (i, 0)),
        pl.BlockSpec((_N, g.shape[1]), lambda i: (0, 0)),
    ] + [pl.BlockSpec(c.shape, lambda i: (0,) * c.ndim) for c in consts]
    return pl.pallas_call(
        body,
        grid=(_N // bm,),
        in_specs=in_specs,
        out_specs=pl.BlockSpec((bm, out_w), lambda i: (i, 0)),
        out_shape=jax.ShapeDtypeStruct((_N, out_w), out_dtype),
        compiler_params=pltpu.CompilerParams(
            dimension_semantics=("parallel",)),
    )(adj, g, *consts)


def _layer1_call(adj, data_b, w1):
    in_specs = [
        pl.BlockSpec((_BM1, _N), lambda i: (i, 0)),
        pl.BlockSpec(data_b.shape, lambda i: (0, 0)),
        pl.BlockSpec(w1.shape, lambda i: (0, 0)),
    ]
    out_specs = [
        pl.BlockSpec((_BM1, 256), lambda i: (i, 0)),
        pl.BlockSpec((_BM1, _N), lambda i: (i, 0)),
    ]
    out_shape = [
        jax.ShapeDtypeStruct((_N, 256), jnp.bfloat16),
        jax.ShapeDtypeStruct((_N, _N), jnp.bfloat16),
    ]
    return pl.pallas_call(
        _layer1_body,
        grid=(_N // _BM1,),
        in_specs=in_specs,
        out_specs=out_specs,
        out_shape=out_shape,
        compiler_params=pltpu.CompilerParams(
            dimension_semantics=("parallel",)),
    )(adj, data_b, w1)


@functools.partial(jax.jit, static_argnames=())
def kernel(data, adj_m, W1, W2, W3, W_out, b_out):
    bf = jnp.bfloat16
    w1 = _pad2(W1, 128, 256).astype(bf)
    w2 = _pad2(W2, 256, 384).astype(bf)
    w3 = _pad2(W3, 384, 256).astype(bf)
    wo = _pad2(W_out, 256, 128).astype(bf)
    b = jnp.pad(b_out, (0, 128 - b_out.shape[0])).reshape(1, 128)
    data_b = data.astype(bf)

    h1, adj_b = _layer1_call(adj_m, data_b, w1)
    g2 = _strip_call(_layer2_body, _BM, adj_b, h1, (w2, w3), 256, bf)
    z = _strip_call(_layer3_body, _BM, adj_b, g2, (wo, b), 128, jnp.float32)
    return z[:, :100]
